# baseline (speedup 1.0000x reference)
import functools

import jax
import jax.numpy as jnp
from jax import lax
from jax.experimental import pallas as pl
from jax.experimental.pallas import tpu as pltpu
from jax.experimental.pallas import tpu_sc as plsc

NUM_CORES = 2
NUM_SUBCORES = 16
NW = NUM_CORES * NUM_SUBCORES

CHUNK = 128


def _build_gather(n_total: int, n_chunks: int, d_model: int):
    mesh = plsc.VectorSubcoreMesh(
        core_axis_name="c", subcore_axis_name="s",
        num_cores=NUM_CORES, num_subcores=NUM_SUBCORES)
    per_w = n_chunks * CHUNK

    @functools.partial(
        pl.kernel,
        out_type=jax.ShapeDtypeStruct((n_total, d_model), jnp.float32),
        mesh=mesh,
        scratch_types=[
            pltpu.VMEM((n_chunks, CHUNK), jnp.int32),
            pltpu.VMEM((CHUNK, d_model), jnp.float32),
            pltpu.SemaphoreType.DMA,
        ],
        compiler_params=pltpu.CompilerParams(use_tc_tiling_on_sc=False),
    )
    def gather_kernel(idx_hbm, table_hbm, out_hbm, idx_v, rows_v, sem):
        wid = lax.axis_index("s") * NUM_CORES + lax.axis_index("c")
        base = wid * per_w
        pltpu.sync_copy(idx_hbm.at[wid], idx_v)

        @pl.loop(0, n_chunks)
        def _(j):
            pltpu.async_copy(table_hbm.at[idx_v.at[j]], rows_v, sem).wait()
            pltpu.sync_copy(rows_v,
                            out_hbm.at[pl.ds(base + j * CHUNK, CHUNK)])

    return gather_kernel


def kernel(x, embedding_matrix):
    batch, seq = x.shape
    _, d_model = embedding_matrix.shape
    n_total = batch * seq
    assert n_total % (NW * CHUNK) == 0
    n_chunks = n_total // (NW * CHUNK)
    idx = x.reshape(NW, n_chunks, CHUNK).astype(jnp.int32)
    gather = _build_gather(n_total, n_chunks, d_model)
    out = gather(idx, embedding_matrix)
    return out.reshape(batch, seq, d_model)


# speedup vs baseline: 1.0460x; 1.0460x over previous
import functools

import jax
import jax.numpy as jnp
from jax import lax
from jax.experimental import pallas as pl
from jax.experimental.pallas import tpu as pltpu
from jax.experimental.pallas import tpu_sc as plsc

NUM_CORES = 2
NUM_SUBCORES = 16
NW = NUM_CORES * NUM_SUBCORES

CHUNK = 128
NBUF = 10


def _build_gather(n_total: int, n_chunks: int, d_model: int):
    mesh = plsc.VectorSubcoreMesh(
        core_axis_name="c", subcore_axis_name="s",
        num_cores=NUM_CORES, num_subcores=NUM_SUBCORES)
    per_w = n_chunks * CHUNK
    n_groups = n_chunks // NBUF

    @functools.partial(
        pl.kernel,
        out_type=jax.ShapeDtypeStruct((n_total, d_model), jnp.float32),
        mesh=mesh,
        scratch_types=[
            pltpu.VMEM((n_chunks, CHUNK), jnp.int32),
            pltpu.VMEM((NBUF, CHUNK, d_model), jnp.float32),
            pltpu.SemaphoreType.DMA,
            pltpu.SemaphoreType.DMA,
        ],
        compiler_params=pltpu.CompilerParams(use_tc_tiling_on_sc=False),
    )
    def gather_kernel(idx_hbm, table_hbm, out_hbm, idx_v, rows_v, gsem, ssem):
        wid = lax.axis_index("s") * NUM_CORES + lax.axis_index("c")
        base = wid * per_w
        pltpu.sync_copy(idx_hbm.at[wid], idx_v)

        def start_gather(j, b):
            pltpu.make_async_copy(
                table_hbm.at[idx_v.at[j]], rows_v.at[b], gsem).start()

        def store_desc(j, b):
            return pltpu.make_async_copy(
                rows_v.at[b], out_hbm.at[pl.ds(base + j * CHUNK, CHUNK)],
                ssem)

        for b in range(NBUF):
            start_gather(b, b)

        @pl.loop(0, n_groups)
        def _(g):
            j0 = g * NBUF
            for b in range(NBUF):
                pltpu.make_async_copy(
                    table_hbm.at[idx_v.at[j0 + b]], rows_v.at[b],
                    gsem).wait()
                store_desc(j0 + b, b).start()
            @pl.when(g + 1 < n_groups)
            def _():
                for b in range(NBUF):
                    store_desc(j0 + b, b).wait()
                    start_gather(j0 + NBUF + b, b)

            @pl.when(g + 1 == n_groups)
            def _():
                for b in range(NBUF):
                    store_desc(j0 + b, b).wait()

    return gather_kernel


def kernel(x, embedding_matrix):
    batch, seq = x.shape
    _, d_model = embedding_matrix.shape
    n_total = batch * seq
    assert n_total % (NW * CHUNK) == 0
    n_chunks = n_total // (NW * CHUNK)
    idx = x.reshape(NW, n_chunks, CHUNK).astype(jnp.int32)
    gather = _build_gather(n_total, n_chunks, d_model)
    out = gather(idx, embedding_matrix)
    return out.reshape(batch, seq, d_model)


# speedup vs baseline: 1.2858x; 1.2293x over previous
import functools

import jax
import jax.numpy as jnp
from jax import lax
from jax.experimental import pallas as pl
from jax.experimental.pallas import tpu as pltpu
from jax.experimental.pallas import tpu_sc as plsc

NUM_CORES = 2
NUM_SUBCORES = 16
NW = NUM_CORES * NUM_SUBCORES

XROWS = 2
NBUF = 8


def _build_gather(batch: int, seq: int, d_model: int):
    mesh = plsc.VectorSubcoreMesh(
        core_axis_name="c", subcore_axis_name="s",
        num_cores=NUM_CORES, num_subcores=NUM_SUBCORES)
    chunk = XROWS * seq
    rows_per_w = batch // NW
    n_chunks = rows_per_w // XROWS
    n_groups = n_chunks // NBUF

    @functools.partial(
        pl.kernel,
        out_type=jax.ShapeDtypeStruct((batch, seq, d_model), jnp.float32),
        mesh=mesh,
        scratch_types=[
            pltpu.VMEM((n_chunks, chunk), jnp.int32),
            pltpu.VMEM((NBUF, chunk, d_model), jnp.float32),
            pltpu.SemaphoreType.DMA,
            pltpu.SemaphoreType.DMA,
        ],
        compiler_params=pltpu.CompilerParams(use_tc_tiling_on_sc=False),
    )
    def gather_kernel(idx_hbm, table_hbm, out_hbm, idx_v, rows_v, gsem, ssem):
        wid = lax.axis_index("s") * NUM_CORES + lax.axis_index("c")
        r0 = wid * rows_per_w
        pltpu.sync_copy(idx_hbm.at[pl.ds(wid * n_chunks, n_chunks)], idx_v)

        def gather_desc(j, b):
            return pltpu.make_async_copy(
                table_hbm.at[idx_v.at[j]], rows_v.at[b], gsem)

        def store_descs(j, b):
            return [
                pltpu.make_async_copy(
                    rows_v.at[b].at[pl.ds(x * seq, seq)],
                    out_hbm.at[r0 + j * XROWS + x], ssem)
                for x in range(XROWS)
            ]

        for b in range(NBUF):
            gather_desc(b, b).start()

        @pl.loop(0, n_groups)
        def _(g):
            j0 = g * NBUF
            for b in range(NBUF):
                gather_desc(j0 + b, b).wait()
                for d in store_descs(j0 + b, b):
                    d.start()
            @pl.when(g + 1 < n_groups)
            def _():
                for b in range(NBUF):
                    for d in store_descs(j0 + b, b):
                        d.wait()
                    gather_desc(j0 + NBUF + b, b).start()

            @pl.when(g + 1 == n_groups)
            def _():
                for b in range(NBUF):
                    for d in store_descs(j0 + b, b):
                        d.wait()

    return gather_kernel


def kernel(x, embedding_matrix):
    batch, seq = x.shape
    _, d_model = embedding_matrix.shape
    gather = _build_gather(batch, seq, d_model)
    idx = x.reshape(batch * seq // (XROWS * seq), XROWS * seq)
    return gather(idx, embedding_matrix)
